# Initial kernel scaffold; baseline (speedup 1.0000x reference)
import jax
import jax.numpy as jnp
from jax.experimental import pallas as pl


def kernel(x, feature, adj, fc_W, fc_b, ln_gamma, ln_beta, W, lstm_Wih, lstm_Whh, lstm_bih, lstm_bhh):
    raise NotImplementedError("write your pallas kernel here")


# speedup vs baseline: 13.0101x; 13.0101x over previous
import jax
import jax.numpy as jnp
from jax import lax
from jax.experimental import pallas as pl
from jax.experimental.pallas import tpu as pltpu

_BSZ, _N, _HIDDEN, _LABELS = 4, 64, 128, 3
_HID2 = _HIDDEN // 2
_ROWS = _BSZ * _N * _LABELS
_G = _LABELS * _BSZ
_GW = 4 * _HIDDEN


def _stats_kernel(madj_ref, out_ref):
    madj = madj_ref[...]
    cnt = jnp.sum(madj, axis=1, keepdims=True)
    tot = cnt[0:256] + cnt[256:512] + cnt[512:768]
    maxv = jnp.max(tot, keepdims=True)
    minv = jnp.min(cnt, keepdims=True)
    lane = lax.broadcasted_iota(jnp.int32, (8, 128), 1)
    out_ref[...] = jnp.where(lane == 0,
                             jnp.broadcast_to(maxv, (8, 128)),
                             jnp.broadcast_to(minv, (8, 128))).astype(jnp.int32)


def _main_kernel(stats_ref, x_ref, ft_ref, madj_ref, fcwx_ref, fcwf_ref,
                 fcb_ref, lng_ref, lnb_ref, w_ref, wih_ref, whh_ref,
                 bias_ref, out_ref, u_scr):
    f32 = jnp.float32
    maxnb = stats_ref[0]
    mincnt = stats_ref[1]

    x = x_ref[...]
    ft = ft_ref[...]
    h0 = jnp.dot(x, fcwx_ref[...], preferred_element_type=f32)
    h0 = h0 + ft[:, 0:1] * fcwf_ref[0:1, :] + ft[:, 1:2] * fcwf_ref[1:2, :]
    h0 = h0 + fcb_ref[...]
    mu = jnp.mean(h0, axis=1, keepdims=True)
    d = h0 - mu
    var = jnp.mean(d * d, axis=1, keepdims=True)
    hf = jnp.tanh(d * lax.rsqrt(var + 1e-5) * lng_ref[...] + lnb_ref[...])

    wih = wih_ref[...]
    whh = whh_ref[...]
    bias3 = bias_ref[...].reshape(1, 1, _GW)
    wall = w_ref[...]
    ubs = []
    for i in range(_LABELS):
        wc = jnp.dot(wall[i], wih, preferred_element_type=f32)
        ubs.append(jnp.dot(hf, wc, preferred_element_type=f32))
    uball = jnp.concatenate(ubs, axis=0)
    u_scr[...] = uball.reshape(_G, _N, _GW)

    madjf = madj_ref[...].astype(f32)
    cnt = jnp.sum(madj_ref[...], axis=1, keepdims=True)
    lane64 = lax.broadcasted_iota(jnp.int32, (_N, 1), 0)

    def gates(z3, c3):
        ig = jax.nn.sigmoid(z3[..., 0 * _HIDDEN:1 * _HIDDEN])
        fg = jax.nn.sigmoid(z3[..., 1 * _HIDDEN:2 * _HIDDEN])
        gg = jnp.tanh(z3[..., 2 * _HIDDEN:3 * _HIDDEN])
        og = jax.nn.sigmoid(z3[..., 3 * _HIDDEN:4 * _HIDDEN])
        c_new = fg * c3 + ig * gg
        h_new = og * jnp.tanh(c_new)
        return h_new, c_new

    def phase1(l, carry):
        h, c = carry
        z = jnp.dot(h, whh, preferred_element_type=f32)
        u = u_scr[:, pl.ds(l, 1), :]
        z3 = z.reshape(_G, _N, _GW) + u + bias3
        h3 = h.reshape(_G, _N, _HIDDEN)
        c3 = c.reshape(_G, _N, _HIDDEN)
        hn, cn = gates(z3, c3)
        onehot = (lane64 == l).astype(f32)
        col = jnp.dot(madjf, onehot, preferred_element_type=f32)
        keep = (col > 0.5).reshape(_G, _N, 1)
        h3 = jnp.where(keep, hn, h3)
        c3 = jnp.where(keep, cn, c3)
        return h3.reshape(_ROWS, _HIDDEN), c3.reshape(_ROWS, _HIDDEN)

    def phase2(t, carry):
        h, c = carry
        z = jnp.dot(h, whh, preferred_element_type=f32)
        z3 = z.reshape(_G, _N, _GW) + bias3
        h3 = h.reshape(_G, _N, _HIDDEN)
        c3 = c.reshape(_G, _N, _HIDDEN)
        hn, cn = gates(z3, c3)
        keep = (cnt <= t).reshape(_G, _N, 1)
        h3 = jnp.where(keep, hn, h3)
        c3 = jnp.where(keep, cn, c3)
        return h3.reshape(_ROWS, _HIDDEN), c3.reshape(_ROWS, _HIDDEN)

    init = (jnp.zeros((_ROWS, _HIDDEN), f32), jnp.zeros((_ROWS, _HIDDEN), f32))
    carry = lax.fori_loop(0, _N, phase1, init)
    carry = lax.fori_loop(mincnt, maxnb, phase2, carry)
    h_t = carry[0]

    mean3 = (h_t[0:256] + h_t[256:512] + h_t[512:768]) * (1.0 / 3.0)
    out_ref[...] = 2.0 * hf + mean3


def kernel(x, feature, adj, fc_W, fc_b, ln_gamma, ln_beta, W,
           lstm_Wih, lstm_Whh, lstm_bih, lstm_bhh):
    madj = adj.transpose(1, 0, 2, 3).reshape(_ROWS, _N)

    stats_full = pl.pallas_call(
        _stats_kernel,
        out_shape=jax.ShapeDtypeStruct((8, 128), jnp.int32),
    )(madj)
    stats = stats_full[0, :2]

    x2 = x.reshape(_BSZ * _N, _HIDDEN)
    ft2 = feature.reshape(_BSZ * _N, 2)
    fcwx = fc_W[:, :_HIDDEN].T
    fcwf = fc_W[:, _HIDDEN:].T
    fcb = fc_b.reshape(1, _HIDDEN)
    lng = ln_gamma.reshape(1, _HIDDEN)
    lnb = ln_beta.reshape(1, _HIDDEN)
    wih_t = lstm_Wih.T
    whh_t = lstm_Whh.T
    bias = (lstm_bih + lstm_bhh).reshape(1, _GW)

    out2 = pl.pallas_call(
        _main_kernel,
        in_specs=[pl.BlockSpec(memory_space=pltpu.SMEM)] + [
            pl.BlockSpec(memory_space=pltpu.VMEM)] * 12,
        out_specs=pl.BlockSpec(memory_space=pltpu.VMEM),
        out_shape=jax.ShapeDtypeStruct((_BSZ * _N, _HIDDEN), jnp.float32),
        scratch_shapes=[pltpu.VMEM((_G, _N, _GW), jnp.float32)],
    )(stats, x2, ft2, madj, fcwx, fcwf, fcb, lng, lnb, W, wih_t, whh_t, bias)
    return out2.reshape(_BSZ, _N, _HIDDEN)


# speedup vs baseline: 14.8036x; 1.1379x over previous
import jax
import jax.numpy as jnp
from jax import lax
from jax.experimental import pallas as pl
from jax.experimental.pallas import tpu as pltpu

_BSZ, _N, _HIDDEN, _LABELS = 4, 64, 128, 3
_HID2 = _HIDDEN // 2
_ROWS = _BSZ * _N * _LABELS
_G = _BSZ * _LABELS
_GW = 4 * _HIDDEN


def _stats_kernel(madj_ref, out_ref):
    madj = madj_ref[...]
    cnt = jnp.sum(madj, axis=1, keepdims=True)
    parts = []
    for j in range(_BSZ):
        b = j * _LABELS * _N
        parts.append(cnt[b:b + _N] + cnt[b + _N:b + 2 * _N] + cnt[b + 2 * _N:b + 3 * _N])
    tot = jnp.concatenate(parts, axis=0)
    maxv = jnp.max(tot, keepdims=True)
    minv = jnp.min(cnt, keepdims=True)
    lane = lax.broadcasted_iota(jnp.int32, (8, 128), 1)
    out_ref[...] = jnp.where(lane == 0,
                             jnp.broadcast_to(maxv, (8, 128)),
                             jnp.broadcast_to(minv, (8, 128))).astype(jnp.int32)


def _main_kernel(stats_ref, x_ref, ft_ref, madj_ref, fcwx_ref, fcwf_ref,
                 fcb_ref, lng_ref, lnb_ref, w_ref, wih_ref, whh_ref,
                 bias_ref, out_ref, u_scr):
    f32 = jnp.float32
    maxnb = stats_ref[0]
    mincnt = stats_ref[1]

    x = x_ref[...]
    ft = ft_ref[...]
    h0 = jnp.dot(x, fcwx_ref[...], preferred_element_type=f32)
    h0 = h0 + ft[:, 0:1] * fcwf_ref[0:1, :] + ft[:, 1:2] * fcwf_ref[1:2, :]
    h0 = h0 + fcb_ref[...]
    mu = jnp.mean(h0, axis=1, keepdims=True)
    d = h0 - mu
    var = jnp.mean(d * d, axis=1, keepdims=True)
    hf = jnp.tanh(d * lax.rsqrt(var + 1e-5) * lng_ref[...] + lnb_ref[...])

    lane512 = lax.broadcasted_iota(jnp.int32, (1, _GW), 1)
    gscale = jnp.where((lane512 >= 2 * _HIDDEN) & (lane512 < 3 * _HIDDEN),
                       jnp.float32(1.0), jnp.float32(0.5))
    wih = wih_ref[...]
    whh_s = whh_ref[...] * gscale
    bias = bias_ref[...]
    bias_s3 = (bias * gscale).reshape(1, 1, _GW)

    wall = w_ref[...]
    for i in range(_LABELS):
        wc = jnp.dot(wall[i], wih, preferred_element_type=f32)
        ub = jnp.dot(hf, wc, preferred_element_type=f32)
        ubs = (ub + bias) * gscale
        for j in range(_BSZ):
            u_scr[j * _LABELS + i] = ubs[j * _N:(j + 1) * _N]

    madjf = madj_ref[...].astype(f32)
    cnt = jnp.sum(madj_ref[...], axis=1, keepdims=True)
    lane64 = lax.broadcasted_iota(jnp.int32, (_N, 1), 0)

    def gates(z3, c3):
        ig = 0.5 + 0.5 * jnp.tanh(z3[..., 0 * _HIDDEN:1 * _HIDDEN])
        fg = 0.5 + 0.5 * jnp.tanh(z3[..., 1 * _HIDDEN:2 * _HIDDEN])
        gg = jnp.tanh(z3[..., 2 * _HIDDEN:3 * _HIDDEN])
        og = 0.5 + 0.5 * jnp.tanh(z3[..., 3 * _HIDDEN:4 * _HIDDEN])
        c_new = fg * c3 + ig * gg
        h_new = og * jnp.tanh(c_new)
        return h_new, c_new

    def phase1(l, carry):
        h, c = carry
        z = jnp.dot(h, whh_s, preferred_element_type=f32)
        u = u_scr[:, pl.ds(l, 1), :]
        z3 = z.reshape(_G, _N, _GW) + u
        h3 = h.reshape(_G, _N, _HIDDEN)
        c3 = c.reshape(_G, _N, _HIDDEN)
        hn, cn = gates(z3, c3)
        onehot = (lane64 == l).astype(f32)
        col = jnp.dot(madjf, onehot, preferred_element_type=f32)
        keep = (col > 0.5).reshape(_G, _N, 1)
        h3 = jnp.where(keep, hn, h3)
        c3 = jnp.where(keep, cn, c3)
        return h3.reshape(_ROWS, _HIDDEN), c3.reshape(_ROWS, _HIDDEN)

    def phase2(t, carry):
        h, c = carry
        z = jnp.dot(h, whh_s, preferred_element_type=f32)
        z3 = z.reshape(_G, _N, _GW) + bias_s3
        h3 = h.reshape(_G, _N, _HIDDEN)
        c3 = c.reshape(_G, _N, _HIDDEN)
        hn, cn = gates(z3, c3)
        keep = (cnt <= t).reshape(_G, _N, 1)
        h3 = jnp.where(keep, hn, h3)
        c3 = jnp.where(keep, cn, c3)
        return h3.reshape(_ROWS, _HIDDEN), c3.reshape(_ROWS, _HIDDEN)

    init = (jnp.zeros((_ROWS, _HIDDEN), f32), jnp.zeros((_ROWS, _HIDDEN), f32))
    carry = lax.fori_loop(0, _N, phase1, init)
    carry = lax.fori_loop(mincnt, maxnb, phase2, carry)
    h_t = carry[0]

    parts = []
    for j in range(_BSZ):
        b = j * _LABELS * _N
        parts.append(h_t[b:b + _N] + h_t[b + _N:b + 2 * _N] + h_t[b + 2 * _N:b + 3 * _N])
    mean3 = jnp.concatenate(parts, axis=0) * (1.0 / 3.0)
    out_ref[...] = 2.0 * hf + mean3


def kernel(x, feature, adj, fc_W, fc_b, ln_gamma, ln_beta, W,
           lstm_Wih, lstm_Whh, lstm_bih, lstm_bhh):
    madj = adj.reshape(_ROWS, _N)

    stats_full = pl.pallas_call(
        _stats_kernel,
        out_shape=jax.ShapeDtypeStruct((8, 128), jnp.int32),
    )(madj)
    stats = stats_full[0, :2]

    x2 = x.reshape(_BSZ * _N, _HIDDEN)
    ft2 = feature.reshape(_BSZ * _N, 2)
    fcwx = fc_W[:, :_HIDDEN].T
    fcwf = fc_W[:, _HIDDEN:].T
    fcb = fc_b.reshape(1, _HIDDEN)
    lng = ln_gamma.reshape(1, _HIDDEN)
    lnb = ln_beta.reshape(1, _HIDDEN)
    wih_t = lstm_Wih.T
    whh_t = lstm_Whh.T
    bias = (lstm_bih + lstm_bhh).reshape(1, _GW)

    out2 = pl.pallas_call(
        _main_kernel,
        in_specs=[pl.BlockSpec(memory_space=pltpu.SMEM)] + [
            pl.BlockSpec(memory_space=pltpu.VMEM)] * 12,
        out_specs=pl.BlockSpec(memory_space=pltpu.VMEM),
        out_shape=jax.ShapeDtypeStruct((_BSZ * _N, _HIDDEN), jnp.float32),
        scratch_shapes=[pltpu.VMEM((_G, _N, _GW), jnp.float32)],
    )(stats, x2, ft2, madj, fcwx, fcwf, fcb, lng, lnb, W, wih_t, whh_t, bias)
    return out2.reshape(_BSZ, _N, _HIDDEN)


# speedup vs baseline: 21.7771x; 1.4711x over previous
import jax
import jax.numpy as jnp
from jax import lax
from jax.experimental import pallas as pl
from jax.experimental.pallas import tpu as pltpu

_BSZ, _N, _HIDDEN, _LABELS = 4, 64, 128, 3
_HID2 = _HIDDEN // 2
_ROWS = _BSZ * _N * _LABELS
_G = _BSZ * _LABELS
_GW = 4 * _HIDDEN


def _stats_kernel(madj_ref, out_ref):
    madj = madj_ref[...]
    cnt = jnp.sum(madj, axis=1, keepdims=True)
    parts = []
    for j in range(_BSZ):
        b = j * _LABELS * _N
        parts.append(cnt[b:b + _N] + cnt[b + _N:b + 2 * _N] + cnt[b + 2 * _N:b + 3 * _N])
    tot = jnp.concatenate(parts, axis=0)
    maxv = jnp.max(tot, keepdims=True)
    minv = jnp.min(cnt, keepdims=True)
    lane = lax.broadcasted_iota(jnp.int32, (8, 128), 1)
    out_ref[...] = jnp.where(lane == 0,
                             jnp.broadcast_to(maxv, (8, 128)),
                             jnp.broadcast_to(minv, (8, 128))).astype(jnp.int32)


def _main_kernel(stats_ref, x_ref, ft_ref, madj_ref, fcwx_ref, fcwf_ref,
                 fcb_ref, lng_ref, lnb_ref, w_ref, wih_ref, whh_ref,
                 bias_ref, out_ref, u_scr, h_scr, waug_scr, c_scr):
    f32 = jnp.float32
    maxnb = stats_ref[0]
    mincnt = stats_ref[1]

    x = x_ref[...]
    ft = ft_ref[...]
    h0 = jnp.dot(x, fcwx_ref[...], preferred_element_type=f32)
    h0 = h0 + ft[:, 0:1] * fcwf_ref[0:1, :] + ft[:, 1:2] * fcwf_ref[1:2, :]
    h0 = h0 + fcb_ref[...]
    mu = jnp.mean(h0, axis=1, keepdims=True)
    d = h0 - mu
    var = jnp.mean(d * d, axis=1, keepdims=True)
    hf = jnp.tanh(d * lax.rsqrt(var + 1e-5) * lng_ref[...] + lnb_ref[...])

    lane512 = lax.broadcasted_iota(jnp.int32, (1, _GW), 1)
    gscale = jnp.where((lane512 >= 2 * _HIDDEN) & (lane512 < 3 * _HIDDEN),
                       jnp.float32(1.0), jnp.float32(0.5))
    wih = wih_ref[...]
    whh_s = whh_ref[...] * gscale
    bias = bias_ref[...]
    bias_s2 = bias * gscale

    wall = w_ref[...]
    for i in range(_LABELS):
        wc = jnp.dot(wall[i], wih, preferred_element_type=f32)
        ub = jnp.dot(hf, wc, preferred_element_type=f32)
        ubs = (ub + bias) * gscale
        for j in range(_BSZ):
            u_scr[:, j * _LABELS + i, :] = ubs[j * _N:(j + 1) * _N]

    lane16 = lax.broadcasted_iota(jnp.int32, (_ROWS, 16), 1)
    gid = lax.broadcasted_iota(jnp.int32, (_ROWS, 16), 0) // _N
    h_scr[:, 0:_HIDDEN] = jnp.zeros((_ROWS, _HIDDEN), f32)
    h_scr[:, _HIDDEN:_HIDDEN + 16] = (lane16 == gid).astype(f32)
    waug_scr[0:_HIDDEN, :] = whh_s
    waug_scr[_HIDDEN:_HIDDEN + 16, :] = jnp.zeros((16, _GW), f32)

    madjf = madj_ref[...].astype(f32)
    cnt = jnp.sum(madj_ref[...], axis=1, keepdims=True)
    lane64 = lax.broadcasted_iota(jnp.int32, (_N, 1), 0)

    def gates(z3, c3):
        ig = 0.5 + 0.5 * jnp.tanh(z3[..., 0 * _HIDDEN:1 * _HIDDEN])
        fg = 0.5 + 0.5 * jnp.tanh(z3[..., 1 * _HIDDEN:2 * _HIDDEN])
        gg = jnp.tanh(z3[..., 2 * _HIDDEN:3 * _HIDDEN])
        og = 0.5 + 0.5 * jnp.tanh(z3[..., 3 * _HIDDEN:4 * _HIDDEN])
        c_new = fg * c3 + ig * gg
        h_new = og * jnp.tanh(c_new)
        return h_new, c_new

    def phase1(l, carry):
        waug_scr[_HIDDEN:_HIDDEN + _G, :] = u_scr[pl.ds(l, 1)].reshape(_G, _GW)
        haug = h_scr[...]
        h = haug[:, 0:_HIDDEN]
        z = jnp.dot(haug, waug_scr[...], preferred_element_type=f32)
        z3 = z.reshape(_G, _N, _GW)
        h3 = h.reshape(_G, _N, _HIDDEN)
        c3 = c_scr[...].reshape(_G, _N, _HIDDEN)
        hn, cn = gates(z3, c3)
        onehot = (lane64 == l).astype(f32)
        col = jnp.dot(madjf, onehot, preferred_element_type=f32)
        keep = (col > 0.5).reshape(_G, _N, 1)
        h3 = jnp.where(keep, hn, h3)
        c3 = jnp.where(keep, cn, c3)
        h_scr[:, 0:_HIDDEN] = h3.reshape(_ROWS, _HIDDEN)
        c_scr[...] = c3.reshape(_ROWS, _HIDDEN)
        return carry

    def step2(t_eff):
        haug = h_scr[...]
        h = haug[:, 0:_HIDDEN]
        z = jnp.dot(haug, waug_scr[...], preferred_element_type=f32)
        z3 = z.reshape(_G, _N, _GW)
        h3 = h.reshape(_G, _N, _HIDDEN)
        c3 = c_scr[...].reshape(_G, _N, _HIDDEN)
        hn, cn = gates(z3, c3)
        keep = (cnt <= t_eff).reshape(_G, _N, 1)
        h3 = jnp.where(keep, hn, h3)
        c3 = jnp.where(keep, cn, c3)
        h_scr[:, 0:_HIDDEN] = h3.reshape(_ROWS, _HIDDEN)
        c_scr[...] = c3.reshape(_ROWS, _HIDDEN)

    def phase2(s, carry):
        t0 = mincnt + 32 * s
        step2(t0)
        for d in range(1, 32):
            td = t0 + d
            step2(jnp.where(td < maxnb, td, -1))
        return carry

    c_scr[...] = jnp.zeros((_ROWS, _HIDDEN), f32)
    lax.fori_loop(0, _N, phase1, 0, unroll=32)
    waug_scr[_HIDDEN:_HIDDEN + _G, :] = jnp.broadcast_to(bias_s2, (_G, _GW))
    nquads = (maxnb - mincnt + 31) // 32
    lax.fori_loop(0, nquads, phase2, 0)
    h_t = h_scr[:, 0:_HIDDEN]

    parts = []
    for j in range(_BSZ):
        b = j * _LABELS * _N
        parts.append(h_t[b:b + _N] + h_t[b + _N:b + 2 * _N] + h_t[b + 2 * _N:b + 3 * _N])
    mean3 = jnp.concatenate(parts, axis=0) * (1.0 / 3.0)
    out_ref[...] = 2.0 * hf + mean3


def kernel(x, feature, adj, fc_W, fc_b, ln_gamma, ln_beta, W,
           lstm_Wih, lstm_Whh, lstm_bih, lstm_bhh):
    madj = adj.reshape(_ROWS, _N)

    stats_full = pl.pallas_call(
        _stats_kernel,
        out_shape=jax.ShapeDtypeStruct((8, 128), jnp.int32),
    )(madj)
    stats = stats_full[0, :2]

    x2 = x.reshape(_BSZ * _N, _HIDDEN)
    ft2 = feature.reshape(_BSZ * _N, 2)
    fcwx = fc_W[:, :_HIDDEN].T
    fcwf = fc_W[:, _HIDDEN:].T
    fcb = fc_b.reshape(1, _HIDDEN)
    lng = ln_gamma.reshape(1, _HIDDEN)
    lnb = ln_beta.reshape(1, _HIDDEN)
    wih_t = lstm_Wih.T
    whh_t = lstm_Whh.T
    bias = (lstm_bih + lstm_bhh).reshape(1, _GW)

    out2 = pl.pallas_call(
        _main_kernel,
        in_specs=[pl.BlockSpec(memory_space=pltpu.SMEM)] + [
            pl.BlockSpec(memory_space=pltpu.VMEM)] * 12,
        out_specs=pl.BlockSpec(memory_space=pltpu.VMEM),
        out_shape=jax.ShapeDtypeStruct((_BSZ * _N, _HIDDEN), jnp.float32),
        scratch_shapes=[pltpu.VMEM((_N, _G, _GW), jnp.float32),
                        pltpu.VMEM((_ROWS, _HIDDEN + 16), jnp.float32),
                        pltpu.VMEM((_HIDDEN + 16, _GW), jnp.float32),
                        pltpu.VMEM((_ROWS, _HIDDEN), jnp.float32)],
    )(stats, x2, ft2, madj, fcwx, fcwf, fcb, lng, lnb, W, wih_t, whh_t, bias)
    return out2.reshape(_BSZ, _N, _HIDDEN)
